# baseline (speedup 1.0000x reference)
import jax
import jax.numpy as jnp
from jax.experimental import pallas as pl
from jax.experimental.pallas import tpu as pltpu

_B = 1
_S = 2048
_D = 768
_E = 8
_K = 2
_F = 128

_ZBLK = _S // _E


def _moe_kernel(x_ref, wr_ref, wg_ref, wu_ref, wd_ref,
                out_ref, aux_ref, a_ref, xb_ref):
    e = pl.program_id(0)

    @pl.when(e == 0)
    def _route():
        logits = jnp.dot(x_ref[...], wr_ref[...],
                         preferred_element_type=jnp.float32)
        xb_ref[...] = x_ref[...].astype(jnp.bfloat16)
        iota_e = jax.lax.broadcasted_iota(jnp.int32, logits.shape, 1)
        m1 = jnp.max(logits, axis=1, keepdims=True)
        e1 = jnp.min(jnp.where(logits == m1, iota_e, _E), axis=1, keepdims=True)
        neg_inf = jnp.float32(-jnp.inf)
        logits2 = jnp.where(iota_e == e1, neg_inf, logits)
        m2 = jnp.max(logits2, axis=1, keepdims=True)
        e2 = jnp.min(jnp.where(logits2 == m2, iota_e, _E), axis=1, keepdims=True)

        w1 = jax.nn.sigmoid(m1 - m2)
        a_ref[...] = jnp.where(iota_e == e1, w1, 0.0) + \
                     jnp.where(iota_e == e2, 1.0 - w1, 0.0)

        sel = (iota_e == e1).astype(jnp.float32) + \
              (iota_e == e2).astype(jnp.float32)
        cnt = jnp.sum(sel, axis=0, keepdims=True)
        ex = jnp.exp(logits - m1)
        probs = ex / jnp.sum(ex, axis=1, keepdims=True)
        psum = jnp.sum(probs, axis=0, keepdims=True)
        aux_ref[0, 0] = jnp.sum(cnt * psum) * (_E * _E) / (_S * _S * _B * _B)

    out_ref[pl.ds(e * _ZBLK, _ZBLK), :] = jnp.zeros((_ZBLK, _D), jnp.float32)

    onehot = (jax.lax.broadcasted_iota(jnp.int32, (_E, 1), 0) == e
              ).astype(jnp.float32)
    a = jnp.dot(a_ref[...], onehot, preferred_element_type=jnp.float32)
    g = jnp.dot(xb_ref[...], wg_ref[0].astype(jnp.bfloat16),
                preferred_element_type=jnp.float32)
    u = jnp.dot(xb_ref[...], wu_ref[0].astype(jnp.bfloat16),
                preferred_element_type=jnp.float32)
    ag = a * g
    h = ag * jax.nn.sigmoid(ag) * (a * u)
    hrow = jnp.sum(h, axis=0, keepdims=True)
    row = jnp.dot(hrow, wd_ref[0], preferred_element_type=jnp.float32)
    out_ref[pl.ds(e, 1), :] = row


@jax.jit
def _moe(x_flat, W_router, W_gate, W_up, W_down):
    out, aux = pl.pallas_call(
        _moe_kernel,
        grid=(_E,),
        in_specs=[
            pl.BlockSpec((_S, _D), lambda e: (0, 0)),
            pl.BlockSpec((_D, _E), lambda e: (0, 0)),
            pl.BlockSpec((1, _D, _F), lambda e: (e, 0, 0)),
            pl.BlockSpec((1, _D, _F), lambda e: (e, 0, 0)),
            pl.BlockSpec((1, _F, _D), lambda e: (e, 0, 0)),
        ],
        out_specs=[
            pl.BlockSpec((_S, _D), lambda e: (0, 0)),
            pl.BlockSpec(memory_space=pltpu.SMEM),
        ],
        out_shape=[
            jax.ShapeDtypeStruct((_S, _D), jnp.float32),
            jax.ShapeDtypeStruct((1, 1), jnp.float32),
        ],
        scratch_shapes=[
            pltpu.VMEM((_S, _E), jnp.float32),
            pltpu.VMEM((_S, _D), jnp.bfloat16),
        ],
    )(x_flat, W_router, W_gate, W_up, W_down)
    return out, aux[0, 0]


def kernel(x, W_router, W_gate, W_up, W_down):
    b, s, d = x.shape
    x_flat = x.reshape(-1, d)
    out, aux = _moe(x_flat, W_router, W_gate, W_up, W_down)
    return out.reshape(b, s, d), aux


# speedup vs baseline: 1.0007x; 1.0007x over previous
import jax
import jax.numpy as jnp
from jax.experimental import pallas as pl
from jax.experimental.pallas import tpu as pltpu

_B = 1
_S = 2048
_D = 768
_E = 8
_K = 2
_F = 128

_ZBLK = _S // _E


def _moe_kernel(x_ref, wr_ref, wg_ref, wu_ref, wd_ref,
                out_ref, aux_ref, a_ref):
    e = pl.program_id(0)

    @pl.when(e == 0)
    def _route():
        logits = jnp.dot(x_ref[...], wr_ref[...],
                         preferred_element_type=jnp.float32)
        iota_e = jax.lax.broadcasted_iota(jnp.int32, logits.shape, 1)
        m1 = jnp.max(logits, axis=1, keepdims=True)
        e1 = jnp.min(jnp.where(logits == m1, iota_e, _E), axis=1, keepdims=True)
        neg_inf = jnp.float32(-jnp.inf)
        logits2 = jnp.where(iota_e == e1, neg_inf, logits)
        m2 = jnp.max(logits2, axis=1, keepdims=True)
        e2 = jnp.min(jnp.where(logits2 == m2, iota_e, _E), axis=1, keepdims=True)

        w1 = jax.nn.sigmoid(m1 - m2)
        a_ref[...] = jnp.where(iota_e == e1, w1, 0.0) + \
                     jnp.where(iota_e == e2, 1.0 - w1, 0.0)

        sel = (iota_e == e1).astype(jnp.float32) + \
              (iota_e == e2).astype(jnp.float32)
        cnt = jnp.sum(sel, axis=0, keepdims=True)
        ex = jnp.exp(logits - m1)
        probs = ex / jnp.sum(ex, axis=1, keepdims=True)
        psum = jnp.sum(probs, axis=0, keepdims=True)
        aux_ref[0, 0] = jnp.sum(cnt * psum) * (_E * _E) / (_S * _S * _B * _B)

    out_ref[pl.ds(e * _ZBLK, _ZBLK), :] = jnp.zeros((_ZBLK, _D), jnp.float32)

    onehot = (jax.lax.broadcasted_iota(jnp.int32, (_E, 1), 0) == e
              ).astype(jnp.float32)
    a = jnp.dot(a_ref[...], onehot, preferred_element_type=jnp.float32)
    g = jnp.dot(x_ref[...], wg_ref[0], preferred_element_type=jnp.float32)
    u = jnp.dot(x_ref[...], wu_ref[0], preferred_element_type=jnp.float32)
    ag = a * g
    h = ag * jax.nn.sigmoid(ag) * (a * u)
    hrow = jnp.sum(h, axis=0, keepdims=True)
    row = jnp.dot(hrow, wd_ref[0], preferred_element_type=jnp.float32)
    out_ref[pl.ds(e, 1), :] = row


@jax.jit
def _moe(x_flat, W_router, W_gate, W_up, W_down):
    out, aux = pl.pallas_call(
        _moe_kernel,
        grid=(_E,),
        in_specs=[
            pl.BlockSpec((_S, _D), lambda e: (0, 0)),
            pl.BlockSpec((_D, _E), lambda e: (0, 0)),
            pl.BlockSpec((1, _D, _F), lambda e: (e, 0, 0)),
            pl.BlockSpec((1, _D, _F), lambda e: (e, 0, 0)),
            pl.BlockSpec((1, _F, _D), lambda e: (e, 0, 0)),
        ],
        out_specs=[
            pl.BlockSpec((_S, _D), lambda e: (0, 0)),
            pl.BlockSpec(memory_space=pltpu.SMEM),
        ],
        out_shape=[
            jax.ShapeDtypeStruct((_S, _D), jnp.float32),
            jax.ShapeDtypeStruct((1, 1), jnp.float32),
        ],
        scratch_shapes=[
            pltpu.VMEM((_S, _E), jnp.float32),
        ],
    )(x_flat, W_router, W_gate, W_up, W_down)
    return out, aux[0, 0]


def kernel(x, W_router, W_gate, W_up, W_down):
    b, s, d = x.shape
    x_flat = x.reshape(-1, d)
    out, aux = _moe(x_flat, W_router, W_gate, W_up, W_down)
    return out.reshape(b, s, d), aux


# speedup vs baseline: 1.0756x; 1.0748x over previous
import jax
import jax.numpy as jnp
from jax.experimental import pallas as pl
from jax.experimental.pallas import tpu as pltpu

_B = 1
_S = 2048
_D = 768
_E = 8
_K = 2
_F = 128

_NB = 8
_T = _S // _NB
_EF = _E * _F


def _moe_kernel(x_ref, wr_ref, wg_ref, wu_ref, wd_ref,
                out_ref, aux_ref, wgu_ref, hsum_ref, cnt_ref, ps_ref):
    j = pl.program_id(0)

    @pl.when(j == 0)
    def _merge_weights():
        for e in range(_E):
            wgu_ref[:, e * _F:(e + 1) * _F] = wg_ref[e].astype(jnp.bfloat16)
            wgu_ref[:, _EF + e * _F:_EF + (e + 1) * _F] = \
                wu_ref[e].astype(jnp.bfloat16)

    logits = jnp.dot(x_ref[...], wr_ref[...],
                     preferred_element_type=jnp.float32)
    iota_e = jax.lax.broadcasted_iota(jnp.int32, logits.shape, 1)
    m1 = jnp.max(logits, axis=1, keepdims=True)
    e1 = jnp.min(jnp.where(logits == m1, iota_e, _E), axis=1, keepdims=True)
    neg_inf = jnp.float32(-jnp.inf)
    logits2 = jnp.where(iota_e == e1, neg_inf, logits)
    m2 = jnp.max(logits2, axis=1, keepdims=True)
    e2 = jnp.min(jnp.where(logits2 == m2, iota_e, _E), axis=1, keepdims=True)

    w1 = jax.nn.sigmoid(m1 - m2)

    sel = (iota_e == e1).astype(jnp.float32) + \
          (iota_e == e2).astype(jnp.float32)
    cnt = jnp.sum(sel, axis=0, keepdims=True)
    ex = jnp.exp(logits - m1)
    probs = ex / jnp.sum(ex, axis=1, keepdims=True)
    psum = jnp.sum(probs, axis=0, keepdims=True)

    @pl.when(j == 0)
    def _init():
        cnt_ref[...] = cnt
        ps_ref[...] = psum

    @pl.when(j > 0)
    def _acc_stats():
        cnt_ref[...] += cnt
        ps_ref[...] += psum

    gu = jnp.dot(x_ref[...].astype(jnp.bfloat16), wgu_ref[...],
                 preferred_element_type=jnp.float32)

    le = jax.lax.broadcasted_iota(jnp.int32, (_T, 2 * _EF), 1)
    le = (le // _F) & (_E - 1)
    a_exp = jnp.where(le == e1, w1, 0.0) + \
            jnp.where(le == e2, 1.0 - w1, 0.0)

    agu = a_exp * gu
    ag = agu[:, :_EF]
    au = agu[:, _EF:]
    h = ag * jax.nn.sigmoid(ag) * au
    hs = jnp.sum(h, axis=0, keepdims=True)

    @pl.when(j == 0)
    def _init_h():
        hsum_ref[...] = hs

    @pl.when(j > 0)
    def _acc_h():
        hsum_ref[...] += hs

    out_ref[...] = jnp.zeros((_T, _D), jnp.float32)

    @pl.when(j == _NB - 1)
    def _finish():
        rows = [jnp.dot(hsum_ref[:, e * _F:(e + 1) * _F], wd_ref[e],
                        preferred_element_type=jnp.float32)
                for e in range(_E)]
        out_ref[0:_E, :] = jnp.concatenate(rows, axis=0)
        aux_ref[0, 0] = jnp.sum(cnt_ref[...] * ps_ref[...]) * \
            (_E * _E) / (_S * _S * _B * _B)


@jax.jit
def _moe(x_flat, W_router, W_gate, W_up, W_down):
    out, aux = pl.pallas_call(
        _moe_kernel,
        grid=(_NB,),
        in_specs=[
            pl.BlockSpec((_T, _D), lambda j: (j, 0)),
            pl.BlockSpec((_D, _E), lambda j: (0, 0)),
            pl.BlockSpec((_E, _D, _F), lambda j: (0, 0, 0)),
            pl.BlockSpec((_E, _D, _F), lambda j: (0, 0, 0)),
            pl.BlockSpec((_E, _F, _D), lambda j: (0, 0, 0)),
        ],
        out_specs=[
            pl.BlockSpec((_T, _D), lambda j: ((j + 1) % _NB, 0)),
            pl.BlockSpec(memory_space=pltpu.SMEM),
        ],
        out_shape=[
            jax.ShapeDtypeStruct((_S, _D), jnp.float32),
            jax.ShapeDtypeStruct((1, 1), jnp.float32),
        ],
        scratch_shapes=[
            pltpu.VMEM((_D, 2 * _EF), jnp.bfloat16),
            pltpu.VMEM((1, _EF), jnp.float32),
            pltpu.VMEM((1, _E), jnp.float32),
            pltpu.VMEM((1, _E), jnp.float32),
        ],
    )(x_flat, W_router, W_gate, W_up, W_down)
    return out, aux[0, 0]


def kernel(x, W_router, W_gate, W_up, W_down):
    b, s, d = x.shape
    x_flat = x.reshape(-1, d)
    out, aux = _moe(x_flat, W_router, W_gate, W_up, W_down)
    return out.reshape(b, s, d), aux
